# Initial kernel scaffold; baseline (speedup 1.0000x reference)
import jax
import jax.numpy as jnp
from jax.experimental import pallas as pl


def kernel(input_tensor, item_size, emb_table):
    raise NotImplementedError("write your pallas kernel here")


# speedup vs baseline: 11.2419x; 11.2419x over previous
import functools

import jax
import jax.numpy as jnp
from jax import lax
from jax.experimental import pallas as pl
from jax.experimental.pallas import tpu as pltpu
from jax.experimental.pallas import tpu_sc as plsc

EMB = 128
NUM_ITEMS = 16384
MAX_SIZE = 20

NC = 2
NS = 16
NW = NC * NS
C = 16
ROWS = C * MAX_SIZE
CPW = NUM_ITEMS // (NW * C)
IPW = NUM_ITEMS // NW
NSPLIT = 4
GLEN = ROWS // NSPLIT
NVREG = EMB // 16


def _vlane_gather(x, idx):
    dnums = lax.GatherDimensionNumbers(
        offset_dims=(), collapsed_slice_dims=(0,), start_index_map=(0,))
    return lax.gather(x, idx[:, None], dnums, slice_sizes=(1,),
                      mode=lax.GatherScatterMode.PROMISE_IN_BOUNDS)


def _sc_body(idx_hbm, len_hbm, table_hbm, out_hbm,
             idx_all, len_all, rows_a, rows_b, row0_v, out_a, out_b,
             sem_ga, sem_gb, sem_oa, sem_ob):
    wid = lax.axis_index("s") * NC + lax.axis_index("c")
    chunk0 = wid * CPW

    pltpu.sync_copy(table_hbm.at[pl.ds(0, 1)], row0_v)
    pltpu.sync_copy(idx_hbm.at[pl.ds(wid * IPW * MAX_SIZE, IPW * MAX_SIZE)],
                    idx_all)
    pltpu.sync_copy(len_hbm.at[pl.ds(wid * IPW, IPW)], len_all)

    def issue(ci, rows_buf, sem):
        for k in range(NSPLIT):
            pltpu.async_copy(
                table_hbm.at[idx_all.at[pl.ds(ci * ROWS + k * GLEN, GLEN)]],
                rows_buf.at[pl.ds(k * GLEN, GLEN)], sem)

    def drain_gather(rows_buf, sem):
        for k in range(NSPLIT):
            pltpu.make_async_copy(
                table_hbm.at[idx_all.at[pl.ds(k * GLEN, GLEN)]],
                rows_buf.at[pl.ds(k * GLEN, GLEN)], sem).wait()

    def prep(ci):
        zc = jnp.zeros((16,), jnp.float32)
        for j in range(MAX_SIZE):
            tok = idx_all[pl.ds(ci * ROWS + j * 16, 16)]
            zc = zc + jnp.where(tok == 0, jnp.float32(1.0), jnp.float32(0.0))
        rcpv = jnp.float32(1.0) / len_all[pl.ds(ci * C, C)].astype(jnp.float32)
        return rcpv, zc * rcpv

    def compute(rows_buf, out_buf, rcpv, zrv):
        def item_body(i, c2):
            bidx = jnp.full((16,), i, jnp.int32)
            a = _vlane_gather(rcpv, bidx)
            b = _vlane_gather(zrv, bidx)
            for v in range(NVREG):
                sl = pl.ds(v * 16, 16)
                acc = rows_buf[i, sl]
                for j in range(1, MAX_SIZE):
                    acc = acc + rows_buf[j * 16 + i, sl]
                out_buf[i, sl] = acc * a - b * row0_v[0, sl]
            return c2
        lax.fori_loop(0, C, item_body, 0, unroll=False)

    def store(ci, out_buf, sem):
        pltpu.async_copy(out_buf, out_hbm.at[pl.ds((chunk0 + ci) * C, C)], sem)

    def drain_store(out_buf, sem):
        pltpu.make_async_copy(out_buf, out_hbm.at[pl.ds(0, C)], sem).wait()

    issue(0, rows_a, sem_ga)

    def pair_body(p, carry):
        ca = 2 * p
        cb = 2 * p + 1
        issue(cb, rows_b, sem_gb)
        rcp_a, zr_a = prep(ca)
        drain_gather(rows_a, sem_ga)

        @pl.when(p > 0)
        def _():
            drain_store(out_a, sem_oa)

        compute(rows_a, out_a, rcp_a, zr_a)
        store(ca, out_a, sem_oa)
        issue(jnp.minimum(ca + 2, CPW - 1), rows_a, sem_ga)

        rcp_b, zr_b = prep(cb)
        drain_gather(rows_b, sem_gb)

        @pl.when(p > 0)
        def _():
            drain_store(out_b, sem_ob)

        compute(rows_b, out_b, rcp_b, zr_b)
        store(cb, out_b, sem_ob)
        return carry

    lax.fori_loop(0, CPW // 2, pair_body, 0, unroll=False)
    drain_gather(rows_a, sem_ga)
    drain_store(out_a, sem_oa)
    drain_store(out_b, sem_ob)


def kernel(input_tensor, item_size, emb_table):
    idx_r = (input_tensor.reshape(NUM_ITEMS // C, C, MAX_SIZE)
             .transpose(0, 2, 1).reshape(-1).astype(jnp.int32))
    lens = item_size.astype(jnp.int32)

    mesh = plsc.VectorSubcoreMesh(core_axis_name="c", subcore_axis_name="s")
    run = functools.partial(
        pl.kernel,
        mesh=mesh,
        out_type=jax.ShapeDtypeStruct((NUM_ITEMS, EMB), jnp.float32),
        scratch_types=[
            pltpu.VMEM((IPW * MAX_SIZE,), jnp.int32),
            pltpu.VMEM((IPW,), jnp.int32),
            pltpu.VMEM((ROWS, EMB), jnp.float32),
            pltpu.VMEM((ROWS, EMB), jnp.float32),
            pltpu.VMEM((1, EMB), jnp.float32),
            pltpu.VMEM((C, EMB), jnp.float32),
            pltpu.VMEM((C, EMB), jnp.float32),
            pltpu.SemaphoreType.DMA,
            pltpu.SemaphoreType.DMA,
            pltpu.SemaphoreType.DMA,
            pltpu.SemaphoreType.DMA,
        ],
    )(_sc_body)
    return run(idx_r, lens, emb_table)
